# Initial kernel scaffold; baseline (speedup 1.0000x reference)
import jax
import jax.numpy as jnp
from jax.experimental import pallas as pl


def kernel(inputs, z_weights):
    raise NotImplementedError("write your pallas kernel here")


# speedup vs baseline: 179.7513x; 179.7513x over previous
import dataclasses
import functools

import jax
import jax.numpy as jnp
from jax import lax
from jax.experimental import pallas as pl
from jax.experimental.pallas import tpu as pltpu
from jax.experimental.pallas import tpu_sc as plsc

_NUM_CORES = 2
_NUM_SUBCORES = 16
_LANES = 16
_NW = _NUM_CORES * _NUM_SUBCORES
_TBL_PAD = 128


def _sc_compiler_params():
    cp = pltpu.CompilerParams()
    if "needs_layout_passes" in pltpu.CompilerParams.__dataclass_fields__:
        cp = dataclasses.replace(cp, needs_layout_passes=False)
    return cp


def _embed_sc(tbl, idx_flat, n, chunk):
    mesh = plsc.VectorSubcoreMesh(
        core_axis_name="c", subcore_axis_name="s",
        num_cores=_NUM_CORES, num_subcores=_NUM_SUBCORES,
    )

    @functools.partial(
        pl.kernel,
        out_type=jax.ShapeDtypeStruct((n,), jnp.float32),
        mesh=mesh,
        scratch_types=[
            pltpu.VMEM((_TBL_PAD,), jnp.float32),
            pltpu.VMEM((chunk,), jnp.int32),
            pltpu.VMEM((chunk,), jnp.float32),
        ],
        compiler_params=_sc_compiler_params(),
    )
    def body(tbl_hbm, idx_hbm, out_hbm, tbl_v, idx_v, out_v):
        wid = lax.axis_index("c") * _NUM_SUBCORES + lax.axis_index("s")
        base = wid * chunk
        pltpu.sync_copy(tbl_hbm, tbl_v)
        pltpu.sync_copy(idx_hbm.at[pl.ds(base, chunk)], idx_v)

        @pl.loop(0, chunk, step=_LANES)
        def _(i):
            idx = idx_v[pl.ds(i, _LANES)]
            out_v[pl.ds(i, _LANES)] = plsc.load_gather(tbl_v, [idx])

        pltpu.sync_copy(out_v, out_hbm.at[pl.ds(base, chunk)])

    return body(tbl, idx_flat)


def kernel(inputs, z_weights):
    b, l = inputs.shape
    n = b * l
    chunk = n // _NW
    tbl = jnp.pad(z_weights[:, 0], (0, _TBL_PAD - z_weights.shape[0]))
    idx_flat = inputs.reshape(n).astype(jnp.int32)
    out = _embed_sc(tbl, idx_flat, n, chunk)
    return out.reshape(b, l, 1)


# speedup vs baseline: 209.8327x; 1.1673x over previous
import dataclasses
import functools

import jax
import jax.numpy as jnp
from jax import lax
from jax.experimental import pallas as pl
from jax.experimental.pallas import tpu as pltpu
from jax.experimental.pallas import tpu_sc as plsc

_NUM_CORES = 2
_NUM_SUBCORES = 16
_LANES = 16
_NW = _NUM_CORES * _NUM_SUBCORES
_TBL_PAD = 128


def _sc_compiler_params():
    cp = pltpu.CompilerParams()
    if "needs_layout_passes" in pltpu.CompilerParams.__dataclass_fields__:
        cp = dataclasses.replace(cp, needs_layout_passes=False)
    return cp


def _embed_sc(tbl, idx_flat, n, chunk):
    mesh = plsc.VectorSubcoreMesh(
        core_axis_name="c", subcore_axis_name="s",
        num_cores=_NUM_CORES, num_subcores=_NUM_SUBCORES,
    )

    @functools.partial(
        pl.kernel,
        out_type=jax.ShapeDtypeStruct((n,), jnp.float32),
        mesh=mesh,
        scratch_types=[
            pltpu.VMEM((_TBL_PAD,), jnp.float32),
            pltpu.VMEM((chunk,), jnp.int32),
            pltpu.VMEM((chunk,), jnp.float32),
        ],
        compiler_params=_sc_compiler_params(),
    )
    def body(tbl_hbm, idx_hbm, out_hbm, tbl_v, idx_v, out_v):
        wid = lax.axis_index("c") * _NUM_SUBCORES + lax.axis_index("s")
        base = wid * chunk
        pltpu.sync_copy(tbl_hbm, tbl_v)
        pltpu.sync_copy(idx_hbm.at[pl.ds(base, chunk)], idx_v)

        @plsc.parallel_loop(0, chunk, step=_LANES, unroll=8)
        def _(i):
            idx = idx_v[pl.ds(i, _LANES)]
            out_v[pl.ds(i, _LANES)] = plsc.load_gather(tbl_v, [idx])

        pltpu.sync_copy(out_v, out_hbm.at[pl.ds(base, chunk)])

    return body(tbl, idx_flat)


def kernel(inputs, z_weights):
    b, l = inputs.shape
    n = b * l
    chunk = n // _NW
    tbl = jnp.pad(z_weights[:, 0], (0, _TBL_PAD - z_weights.shape[0]))
    idx_flat = inputs.reshape(n).astype(jnp.int32)
    out = _embed_sc(tbl, idx_flat, n, chunk)
    return out.reshape(b, l, 1)


# speedup vs baseline: 260.6879x; 1.2424x over previous
import dataclasses
import functools

import jax
import jax.numpy as jnp
from jax import lax
from jax.experimental import pallas as pl
from jax.experimental.pallas import tpu as pltpu
from jax.experimental.pallas import tpu_sc as plsc

_NUM_CORES = 2
_NUM_SUBCORES = 16
_LANES = 16
_NW = _NUM_CORES * _NUM_SUBCORES
_TBL_PAD = 128


def _sc_compiler_params():
    cp = pltpu.CompilerParams()
    if "needs_layout_passes" in pltpu.CompilerParams.__dataclass_fields__:
        cp = dataclasses.replace(cp, needs_layout_passes=False)
    return cp


def _embed_sc(tbl, inputs):
    b, l = inputs.shape
    rows = b // _NW
    offs = list(range(0, l - _LANES + 1, _LANES))
    if offs[-1] != l - _LANES:
        offs.append(l - _LANES)

    mesh = plsc.VectorSubcoreMesh(
        core_axis_name="c", subcore_axis_name="s",
        num_cores=_NUM_CORES, num_subcores=_NUM_SUBCORES,
    )

    @functools.partial(
        pl.kernel,
        out_type=jax.ShapeDtypeStruct((b, l), jnp.float32),
        mesh=mesh,
        scratch_types=[
            pltpu.VMEM((_TBL_PAD,), jnp.float32),
            pltpu.VMEM((rows, l), jnp.int32),
            pltpu.VMEM((rows, l), jnp.float32),
        ],
        compiler_params=_sc_compiler_params(),
    )
    def body(tbl_hbm, idx_hbm, out_hbm, tbl_v, idx_v, out_v):
        wid = lax.axis_index("c") * _NUM_SUBCORES + lax.axis_index("s")
        r0 = wid * rows
        pltpu.sync_copy(tbl_hbm, tbl_v)
        pltpu.sync_copy(idx_hbm.at[pl.ds(r0, rows)], idx_v)

        @plsc.parallel_loop(0, rows, step=1, unroll=2)
        def _(r):
            for c in offs:
                idx = idx_v[r, pl.ds(c, _LANES)]
                out_v[r, pl.ds(c, _LANES)] = plsc.load_gather(tbl_v, [idx])

        pltpu.sync_copy(out_v, out_hbm.at[pl.ds(r0, rows)])

    return body(tbl, inputs)


def kernel(inputs, z_weights):
    tbl = jnp.pad(z_weights[:, 0], (0, _TBL_PAD - z_weights.shape[0]))
    return _embed_sc(tbl, inputs.astype(jnp.int32))[..., None]
